# baseline (speedup 1.0000x reference)
import functools

import jax
import jax.numpy as jnp
from jax import lax
from jax.experimental import pallas as pl
from jax.experimental.pallas import tpu as pltpu
from jax.experimental.pallas import tpu_sc as plsc

I_N, J_F, K_T, DIM, LAYERS, HEADS = 10000, 16, 256, 128, 2, 2
E_N, T_N = 320000, 65536
HD = DIM // HEADS

E_PAD = 327680
N1 = E_PAD // (16 * 128)
N2 = E_PAD // (32 * 128)
RH = I_N // 2
RHP = 5008
AFLAT = RHP * K_T
DUMMY_A = RH * K_T
ACHUNK = AFLAT // 16
EXI_R = 10240
EXK_R = 512

_mesh = plsc.VectorSubcoreMesh(core_axis_name="c", subcore_axis_name="s")
_sc_params = pltpu.CompilerParams(use_tc_tiling_on_sc=False,
                                  needs_layout_passes=False)


@functools.partial(
    pl.kernel,
    out_type=[
        jax.ShapeDtypeStruct((2 * AFLAT,), jnp.float32),
        jax.ShapeDtypeStruct((2 * EXI_R, J_F), jnp.float32),
        jax.ShapeDtypeStruct((2 * EXK_R, J_F), jnp.float32),
    ],
    mesh=_mesh,
    compiler_params=_sc_params,
    scratch_types=[
        pltpu.VMEM_SHARED((AFLAT,), jnp.float32),
        pltpu.VMEM_SHARED((EXI_R, J_F), jnp.float32),
        pltpu.VMEM_SHARED((EXK_R, J_F), jnp.float32),
        pltpu.VMEM((16, 128), jnp.int32),
        pltpu.VMEM((16, 128), jnp.int32),
        pltpu.VMEM((128,), jnp.int32),
        pltpu.VMEM((128,), jnp.float32),
        pltpu.VMEM((128, J_F), jnp.float32),
        pltpu.VMEM((ACHUNK // 8, ), jnp.float32),
        pltpu.VMEM((EXI_R // 16, J_F), jnp.float32),
    ],
)
def _sc_build(ei2d, ek2d, ex_p, z1d, z2d, out_a, out_exi, out_exk,
              a_sh, exi_sh, exk_sh, ei_v, ek_v, idx_v, ones_v, ex_v,
              bnc, bexi):
    c = lax.axis_index("c")
    s = lax.axis_index("s")
    wid = s * 2 + c

    pltpu.sync_copy(z1d, bnc)
    pltpu.sync_copy(z2d, bexi)
    for t in range(8):
        pltpu.sync_copy(bnc, a_sh.at[pl.ds(s * ACHUNK + t * (ACHUNK // 8),
                                           ACHUNK // 8)])
    pltpu.sync_copy(bexi, exi_sh.at[pl.ds(s * (EXI_R // 16), EXI_R // 16)])

    @pl.when(s == 0)
    def _():
        pltpu.sync_copy(bexi.at[pl.ds(0, EXK_R)], exk_sh.at[pl.ds(0, EXK_R)])

    for m in range(8):
        ones_v[pl.ds(m * 16, 16)] = jnp.ones((16,), jnp.float32)
    plsc.subcore_barrier()

    rowbase = c * RH

    def p1_outer(oc, carry):
        pltpu.sync_copy(ei2d.at[pl.ds(s * N1 + oc * 16, 16)], ei_v)
        pltpu.sync_copy(ek2d.at[pl.ds(s * N1 + oc * 16, 16)], ek_v)

        def p1_body(j, carry2):
            for m in range(8):
                vi = ei_v[j, pl.ds(m * 16, 16)]
                vk = ek_v[j, pl.ds(m * 16, 16)]
                loc = vi - rowbase
                ok = (loc >= 0) & (loc < RH)
                idx_v[pl.ds(m * 16, 16)] = jnp.where(ok, loc * K_T + vk,
                                                     DUMMY_A)
            pltpu.sync_copy(ones_v, a_sh.at[idx_v], add=True)
            return carry2

        lax.fori_loop(0, 16, p1_body, 0)
        return carry

    lax.fori_loop(0, N1 // 16, p1_outer, 0)

    def p2_outer(oc, carry):
        pltpu.sync_copy(ei2d.at[pl.ds(wid * N2 + oc * 16, 16)], ei_v)
        pltpu.sync_copy(ek2d.at[pl.ds(wid * N2 + oc * 16, 16)], ek_v)

        def p2_body(j, carry2):
            base = (wid * N2 + oc * 16 + j) * 128
            pltpu.sync_copy(ex_p.at[pl.ds(base, 128)], ex_v)
            pltpu.sync_copy(ex_v, exi_sh.at[ei_v.at[j]], add=True)
            pltpu.sync_copy(ex_v, exk_sh.at[ek_v.at[j]], add=True)
            return carry2

        lax.fori_loop(0, 16, p2_body, 0)
        return carry

    lax.fori_loop(0, N2 // 16, p2_outer, 0)
    plsc.subcore_barrier()

    for t in range(8):
        off = s * ACHUNK + t * (ACHUNK // 8)
        pltpu.sync_copy(a_sh.at[pl.ds(off, ACHUNK // 8)], bnc)
        pltpu.sync_copy(bnc, out_a.at[pl.ds(c * AFLAT + off, ACHUNK // 8)])
    pltpu.sync_copy(exi_sh.at[pl.ds(s * (EXI_R // 16), EXI_R // 16)], bexi)
    pltpu.sync_copy(bexi,
                    out_exi.at[pl.ds(c * EXI_R + s * (EXI_R // 16), EXI_R // 16)])

    @pl.when(s == 0)
    def _():
        pltpu.sync_copy(exk_sh.at[pl.ds(0, EXK_R)], bexi.at[pl.ds(0, EXK_R)])
        pltpu.sync_copy(bexi.at[pl.ds(0, EXK_R)],
                        out_exk.at[pl.ds(c * EXK_R, EXK_R)])


def _mmt(x, w):
    return lax.dot_general(x, w, (((1,), (1,)), ((), ())),
                           preferred_element_type=jnp.float32)


def _layer_norm(x, g, b):
    m = jnp.mean(x, axis=-1, keepdims=True)
    v = jnp.mean((x - m) ** 2, axis=-1, keepdims=True)
    return (x - m) * lax.rsqrt(v + 1e-5) * g + b


def _tc_body(a_ref, exi0_ref, exi1_ref, exk0_ref, exk1_ref, u0_ref, w0_ref,
             epw_ref, epb_ref, uupw_ref, uupb_ref, wupw_ref, wupb_ref,
             pos_ref, inw_ref, inb_ref, outw_ref, outb_ref, f1w_ref, f1b_ref,
             f2w_ref, f2b_ref, l1g_ref, l1b_ref, l2g_ref, l2b_ref, v_ref,
             u_out_ref, p_out_ref):
    A = a_ref[...]
    exi = exi0_ref[...] + exi1_ref[...]
    exk = exk0_ref[...] + exk1_ref[...]
    cnt_u_raw = jnp.sum(A, axis=1, keepdims=True)
    cnt_w_raw = jnp.sum(A, axis=0)[:, None]
    cnt_u = jnp.maximum(cnt_u_raw, 1.0)
    cnt_w = jnp.maximum(cnt_w_raw, 1.0)
    epb = epb_ref[...]
    eproj_i = _mmt(exi, epw_ref[...]) + cnt_u_raw * epb[None, :]
    eproj_k = _mmt(exk, epw_ref[...]) + cnt_w_raw * epb[None, :]

    U = u0_ref[...]
    W = w0_ref[...]
    for l in range(LAYERS):
        u_msg = lax.dot_general(A, W, (((1,), (0,)), ((), ())),
                                preferred_element_type=jnp.float32) + eproj_i
        w_msg = lax.dot_general(A, U, (((0,), (0,)), ((), ())),
                                preferred_element_type=jnp.float32) + eproj_k
        uw = uupw_ref[l]
        ww = wupw_ref[l]
        U = jax.nn.relu(_mmt(U, uw[:, :DIM]) + _mmt(u_msg / cnt_u, uw[:, DIM:])
                        + uupb_ref[l][None, :])
        W = jax.nn.relu(_mmt(W, ww[:, :DIM]) + _mmt(w_msg / cnt_w, ww[:, DIM:])
                        + wupb_ref[l][None, :])

    X = W + pos_ref[...]
    qkv = _mmt(X, inw_ref[...]) + inb_ref[...][None, :]
    aos = []
    for h in range(HEADS):
        qh = qkv[:, h * HD:(h + 1) * HD]
        kh = qkv[:, DIM + h * HD:DIM + (h + 1) * HD]
        vh = qkv[:, 2 * DIM + h * HD:2 * DIM + (h + 1) * HD]
        logits = _mmt(qh, kh) * (1.0 / (HD ** 0.5))
        logits = logits - jnp.max(logits, axis=-1, keepdims=True)
        e = jnp.exp(logits)
        attn = e / jnp.sum(e, axis=-1, keepdims=True)
        aos.append(lax.dot_general(attn, vh, (((1,), (0,)), ((), ())),
                                   preferred_element_type=jnp.float32))
    ao = _mmt(jnp.concatenate(aos, axis=1), outw_ref[...]) + outb_ref[...][None, :]
    H1 = _layer_norm(X + ao, l1g_ref[...], l1b_ref[...])
    H2 = _mmt(jax.nn.relu(_mmt(H1, f1w_ref[...]) + f1b_ref[...][None, :]),
              f2w_ref[...]) + f2b_ref[...][None, :]
    Wr = _layer_norm(H1 + H2, l2g_ref[...], l2b_ref[...])

    u_out_ref[...] = U
    p_out_ref[...] = (v_ref[...][:, None, :] * Wr[None, :, :]).reshape(
        J_F * K_T, DIM)


_tc_dense = pl.pallas_call(
    _tc_body,
    out_shape=[
        jax.ShapeDtypeStruct((I_N, DIM), jnp.float32),
        jax.ShapeDtypeStruct((J_F * K_T, DIM), jnp.float32),
    ],
)


@functools.partial(
    pl.kernel,
    out_type=jax.ShapeDtypeStruct((T_N,), jnp.float32),
    mesh=_mesh,
    compiler_params=_sc_params,
    scratch_types=[
        pltpu.VMEM((128,), jnp.int32),
        pltpu.VMEM((128,), jnp.int32),
        pltpu.VMEM((128,), jnp.int32),
        pltpu.VMEM((128,), jnp.int32),
        pltpu.VMEM((128, DIM), jnp.float32),
        pltpu.VMEM((128, DIM), jnp.float32),
        pltpu.VMEM((128,), jnp.float32),
        pltpu.VMEM((16,), jnp.float32),
    ],
)
def _sc_decode(u_hbm, p_hbm, ti_hbm, tj_hbm, tk_hbm, b16, out,
               iv, jv, kv, pidx, urows, prows, outb, bias_v):
    c = lax.axis_index("c")
    s = lax.axis_index("s")
    wid = s * 2 + c
    nchunk = T_N // (32 * 128)
    pltpu.sync_copy(b16, bias_v)
    bvec = bias_v[...]

    def chunk_body(ch, carry):
        base = (wid * nchunk + ch) * 128
        pltpu.sync_copy(ti_hbm.at[pl.ds(base, 128)], iv)
        pltpu.sync_copy(tj_hbm.at[pl.ds(base, 128)], jv)
        pltpu.sync_copy(tk_hbm.at[pl.ds(base, 128)], kv)
        for m in range(8):
            pidx[pl.ds(m * 16, 16)] = (jv[pl.ds(m * 16, 16)] * K_T
                                       + kv[pl.ds(m * 16, 16)])
        pltpu.sync_copy(u_hbm.at[iv], urows)
        pltpu.sync_copy(p_hbm.at[pidx], prows)
        for g in range(8):
            rowv = lax.iota(jnp.int32, 16) + g * 16

            def col_body(c2, acc):
                for u in range(8):
                    colv = jnp.zeros((16,), jnp.int32) + (c2 * 8 + u)
                    uv = plsc.load_gather(urows, [rowv, colv])
                    pv = plsc.load_gather(prows, [rowv, colv])
                    acc = acc + uv * pv
                return acc

            acc = lax.fori_loop(0, DIM // 8, col_body,
                                jnp.zeros((16,), jnp.float32))
            x = acc + bvec
            outb[pl.ds(g * 16, 16)] = 1.0 / (1.0 + jnp.exp(-x))
        pltpu.sync_copy(outb, out.at[pl.ds(base, 128)])
        return carry

    lax.fori_loop(0, nchunk, chunk_body, 0)


def kernel(edge_index, edge_x, idx_ijk, u0_weight, w0, edge_proj_W,
           edge_proj_b, u_up_W, u_up_b, w_up_W, w_up_b, pos, in_proj_W,
           in_proj_b, out_proj_W, out_proj_b, ffn_W1, ffn_b1, ffn_W2,
           ffn_b2, ln1_g, ln1_b, ln2_g, ln2_b, V_weight, bias):
    pad = E_PAD - E_N
    ei = edge_index[0].astype(jnp.int32)
    ek = edge_index[1].astype(jnp.int32)
    ei2d = jnp.concatenate([ei, jnp.full((pad,), I_N, jnp.int32)]).reshape(
        E_PAD // 128, 128)
    ek2d = jnp.concatenate([ek, jnp.full((pad,), K_T, jnp.int32)]).reshape(
        E_PAD // 128, 128)
    ex_p = jnp.concatenate([edge_x, jnp.zeros((pad, J_F), jnp.float32)])
    z1d = jnp.zeros((ACHUNK // 8,), jnp.float32)
    z2d = jnp.zeros((EXI_R // 16, J_F), jnp.float32)

    out_a, out_exi, out_exk = _sc_build(ei2d, ek2d, ex_p, z1d, z2d)
    A = out_a.reshape(2, RHP, K_T)[:, :RH, :].reshape(I_N, K_T)
    exi0 = out_exi[0:I_N]
    exi1 = out_exi[EXI_R:EXI_R + I_N]
    exk0 = out_exk[0:K_T]
    exk1 = out_exk[EXK_R:EXK_R + K_T]

    U, P = _tc_dense(A, exi0, exi1, exk0, exk1, u0_weight, w0, edge_proj_W,
                     edge_proj_b, u_up_W, u_up_b, w_up_W, w_up_b, pos,
                     in_proj_W, in_proj_b, out_proj_W, out_proj_b, ffn_W1,
                     ffn_b1, ffn_W2, ffn_b2, ln1_g, ln1_b, ln2_g, ln2_b,
                     V_weight)

    ti = idx_ijk[:, 0].astype(jnp.int32)
    tj = idx_ijk[:, 1].astype(jnp.int32)
    tk = idx_ijk[:, 2].astype(jnp.int32)
    b16 = jnp.broadcast_to(bias.astype(jnp.float32), (16,))
    return _sc_decode(U, P, ti, tj, tk, b16)


# speedup vs baseline: 1.4617x; 1.4617x over previous
import functools

import jax
import jax.numpy as jnp
from jax import lax
from jax.experimental import pallas as pl
from jax.experimental.pallas import tpu as pltpu
from jax.experimental.pallas import tpu_sc as plsc

I_N, J_F, K_T, DIM, LAYERS, HEADS = 10000, 16, 256, 128, 2, 2
E_N, T_N = 320000, 65536
HD = DIM // HEADS

E_PAD = 327680
N1 = E_PAD // (16 * 128)
N2 = E_PAD // (32 * 128)
RH = I_N // 2
RHP = 5008
AFLAT = RHP * K_T
DUMMY_A = RH * K_T
ACHUNK = AFLAT // 16
EXI_R = 10240
EXK_R = 512

_mesh = plsc.VectorSubcoreMesh(core_axis_name="c", subcore_axis_name="s")
_sc_params = pltpu.CompilerParams(use_tc_tiling_on_sc=False,
                                  needs_layout_passes=False)


NR2 = E_N // 128


@functools.partial(
    pl.kernel,
    out_type=[
        jax.ShapeDtypeStruct((2 * AFLAT,), jnp.float32),
        jax.ShapeDtypeStruct((2 * EXI_R, J_F), jnp.float32),
    ],
    mesh=_mesh,
    compiler_params=_sc_params,
    scratch_types=[
        pltpu.VMEM_SHARED((AFLAT,), jnp.float32),
        pltpu.VMEM_SHARED((EXI_R, J_F), jnp.float32),
        pltpu.VMEM((4, 128), jnp.int32),
        pltpu.VMEM((4, 128), jnp.int32),
        pltpu.VMEM((4, 128), jnp.int32),
        pltpu.VMEM((128,), jnp.float32),
        pltpu.VMEM((4, 128, J_F), jnp.float32),
        pltpu.VMEM((ACHUNK // 8, ), jnp.float32),
        pltpu.VMEM((EXI_R // 16, J_F), jnp.float32),
        pltpu.SemaphoreType.DMA,
        pltpu.SemaphoreType.DMA,
        pltpu.SemaphoreType.DMA,
        pltpu.SemaphoreType.DMA,
        pltpu.SemaphoreType.DMA,
        pltpu.SemaphoreType.DMA,
        pltpu.SemaphoreType.DMA,
        pltpu.SemaphoreType.DMA,
    ],
)
def _sc_build(ei2d, ek2d, ex, z1d, z2d, out_a, out_exi,
              a_sh, exi_sh, eiv, ekv, idx, ones_v, exv, bnc, bexi,
              is0, is1, is2, is3, ss0, ss1, ss2, ss3):
    c = lax.axis_index("c")
    s = lax.axis_index("s")
    wid = s * 2 + c
    isems = (is0, is1, is2, is3)
    ssems = (ss0, ss1, ss2, ss3)

    pltpu.sync_copy(z1d, bnc)
    pltpu.sync_copy(z2d, bexi)
    for t in range(8):
        pltpu.sync_copy(bnc, a_sh.at[pl.ds(s * ACHUNK + t * (ACHUNK // 8),
                                           ACHUNK // 8)])
    pltpu.sync_copy(bexi, exi_sh.at[pl.ds(s * (EXI_R // 16), EXI_R // 16)])
    for m in range(8):
        ones_v[pl.ds(m * 16, 16)] = jnp.ones((16,), jnp.float32)
    plsc.subcore_barrier()

    rowbase = c * RH

    def p1_body(t4, carry):
        rb = s * N1 + t4 * 4
        for par in range(4):
            pltpu.async_copy(ei2d.at[rb + par], eiv.at[par], isems[par])
            pltpu.async_copy(ek2d.at[rb + par], ekv.at[par], isems[par])
        for par in range(4):
            pltpu.make_async_copy(ei2d.at[rb + par], eiv.at[par],
                                  isems[par]).wait()
            pltpu.make_async_copy(ek2d.at[rb + par], ekv.at[par],
                                  isems[par]).wait()
            for m in range(8):
                vi = eiv[par, pl.ds(m * 16, 16)]
                vk = ekv[par, pl.ds(m * 16, 16)]
                loc = vi - rowbase
                ok = (loc >= 0) & (loc < RH)
                idx[par, pl.ds(m * 16, 16)] = jnp.where(ok, loc * K_T + vk,
                                                        DUMMY_A)
            pltpu.async_copy(ones_v, a_sh.at[idx.at[par]], ssems[par],
                             add=True)
        for par in range(4):
            pltpu.make_async_copy(ones_v, a_sh.at[idx.at[par]],
                                  ssems[par]).wait()
        return carry

    lax.fori_loop(0, N1 // 4, p1_body, 0)

    def p2_body(t4, carry):
        rb = wid * N2 + t4 * 4
        for par in range(4):
            @pl.when(rb + par < NR2)
            def _():
                pltpu.async_copy(ei2d.at[rb + par], eiv.at[par], isems[par])
                pltpu.async_copy(ex.at[pl.ds((rb + par) * 128, 128)],
                                 exv.at[par], isems[par])
        for par in range(4):
            @pl.when(rb + par < NR2)
            def _():
                pltpu.make_async_copy(ei2d.at[rb + par], eiv.at[par],
                                      isems[par]).wait()
                pltpu.make_async_copy(ex.at[pl.ds((rb + par) * 128, 128)],
                                      exv.at[par], isems[par]).wait()
                pltpu.async_copy(exv.at[par], exi_sh.at[eiv.at[par]],
                                 ssems[par], add=True)
        for par in range(4):
            @pl.when(rb + par < NR2)
            def _():
                pltpu.make_async_copy(exv.at[par], exi_sh.at[eiv.at[par]],
                                      ssems[par]).wait()
        return carry

    lax.fori_loop(0, N2 // 4, p2_body, 0)
    plsc.subcore_barrier()

    for t in range(8):
        off = s * ACHUNK + t * (ACHUNK // 8)
        pltpu.sync_copy(a_sh.at[pl.ds(off, ACHUNK // 8)], bnc)
        pltpu.sync_copy(bnc, out_a.at[pl.ds(c * AFLAT + off, ACHUNK // 8)])
    pltpu.sync_copy(exi_sh.at[pl.ds(s * (EXI_R // 16), EXI_R // 16)], bexi)
    pltpu.sync_copy(bexi,
                    out_exi.at[pl.ds(c * EXI_R + s * (EXI_R // 16), EXI_R // 16)])


_EXK_CH = 2560


def _exk_body(ek_ref, ex_ref, o_ref):
    i = pl.program_id(0)
    oh = (lax.broadcasted_iota(jnp.int32, (_EXK_CH, K_T), 1)
          == ek_ref[...]).astype(jnp.float32)
    acc = lax.dot_general(oh, ex_ref[...], (((0,), (0,)), ((), ())),
                          preferred_element_type=jnp.float32)

    @pl.when(i == 0)
    def _():
        o_ref[...] = acc

    @pl.when(i > 0)
    def _():
        o_ref[...] += acc


_exk_call = pl.pallas_call(
    _exk_body,
    grid=(E_N // _EXK_CH,),
    in_specs=[
        pl.BlockSpec((_EXK_CH, 1), lambda i: (i, 0)),
        pl.BlockSpec((_EXK_CH, J_F), lambda i: (i, 0)),
    ],
    out_specs=pl.BlockSpec((K_T, J_F), lambda i: (0, 0)),
    out_shape=jax.ShapeDtypeStruct((K_T, J_F), jnp.float32),
)


def _mmt(x, w):
    return lax.dot_general(x, w, (((1,), (1,)), ((), ())),
                           preferred_element_type=jnp.float32)


def _layer_norm(x, g, b):
    m = jnp.mean(x, axis=-1, keepdims=True)
    v = jnp.mean((x - m) ** 2, axis=-1, keepdims=True)
    return (x - m) * lax.rsqrt(v + 1e-5) * g + b


def _tc_body(a_ref, exi0_ref, exi1_ref, exk_ref, u0_ref, w0_ref,
             epw_ref, epb_ref, uupw_ref, uupb_ref, wupw_ref, wupb_ref,
             pos_ref, inw_ref, inb_ref, outw_ref, outb_ref, f1w_ref, f1b_ref,
             f2w_ref, f2b_ref, l1g_ref, l1b_ref, l2g_ref, l2b_ref, v_ref,
             u_out_ref, p_out_ref):
    A = jnp.concatenate([a_ref[0, :RH, :], a_ref[1, :RH, :]], axis=0)
    exi = exi0_ref[...] + exi1_ref[...]
    exk = exk_ref[...]
    cnt_u_raw = jnp.sum(A, axis=1, keepdims=True)
    cnt_w_raw = jnp.sum(A, axis=0)[:, None]
    cnt_u = jnp.maximum(cnt_u_raw, 1.0)
    cnt_w = jnp.maximum(cnt_w_raw, 1.0)
    epb = epb_ref[...]
    eproj_i = _mmt(exi, epw_ref[...]) + cnt_u_raw * epb[None, :]
    eproj_k = _mmt(exk, epw_ref[...]) + cnt_w_raw * epb[None, :]

    U = u0_ref[...]
    W = w0_ref[...]
    for l in range(LAYERS):
        u_msg = lax.dot_general(A, W, (((1,), (0,)), ((), ())),
                                preferred_element_type=jnp.float32) + eproj_i
        w_msg = lax.dot_general(A, U, (((0,), (0,)), ((), ())),
                                preferred_element_type=jnp.float32) + eproj_k
        uw = uupw_ref[l]
        ww = wupw_ref[l]
        U = jax.nn.relu(_mmt(U, uw[:, :DIM]) + _mmt(u_msg / cnt_u, uw[:, DIM:])
                        + uupb_ref[l][None, :])
        W = jax.nn.relu(_mmt(W, ww[:, :DIM]) + _mmt(w_msg / cnt_w, ww[:, DIM:])
                        + wupb_ref[l][None, :])

    X = W + pos_ref[...]
    qkv = _mmt(X, inw_ref[...]) + inb_ref[...][None, :]
    aos = []
    for h in range(HEADS):
        qh = qkv[:, h * HD:(h + 1) * HD]
        kh = qkv[:, DIM + h * HD:DIM + (h + 1) * HD]
        vh = qkv[:, 2 * DIM + h * HD:2 * DIM + (h + 1) * HD]
        logits = _mmt(qh, kh) * (1.0 / (HD ** 0.5))
        logits = logits - jnp.max(logits, axis=-1, keepdims=True)
        e = jnp.exp(logits)
        attn = e / jnp.sum(e, axis=-1, keepdims=True)
        aos.append(lax.dot_general(attn, vh, (((1,), (0,)), ((), ())),
                                   preferred_element_type=jnp.float32))
    ao = _mmt(jnp.concatenate(aos, axis=1), outw_ref[...]) + outb_ref[...][None, :]
    H1 = _layer_norm(X + ao, l1g_ref[...], l1b_ref[...])
    H2 = _mmt(jax.nn.relu(_mmt(H1, f1w_ref[...]) + f1b_ref[...][None, :]),
              f2w_ref[...]) + f2b_ref[...][None, :]
    Wr = _layer_norm(H1 + H2, l2g_ref[...], l2b_ref[...])

    u_out_ref[...] = U
    p_out_ref[...] = (v_ref[...][:, None, :] * Wr[None, :, :]).reshape(
        J_F * K_T, DIM)


_tc_dense = pl.pallas_call(
    _tc_body,
    out_shape=[
        jax.ShapeDtypeStruct((I_N, DIM), jnp.float32),
        jax.ShapeDtypeStruct((J_F * K_T, DIM), jnp.float32),
    ],
)


@functools.partial(
    pl.kernel,
    out_type=[
        jax.ShapeDtypeStruct((T_N, DIM), jnp.float32),
        jax.ShapeDtypeStruct((T_N, DIM), jnp.float32),
    ],
    mesh=_mesh,
    compiler_params=_sc_params,
    scratch_types=[
        pltpu.VMEM((2, 128), jnp.int32),
        pltpu.VMEM((128,), jnp.int32),
        pltpu.VMEM((128,), jnp.int32),
        pltpu.VMEM((2, 128), jnp.int32),
        pltpu.VMEM((2, 128, DIM), jnp.float32),
        pltpu.VMEM((2, 128, DIM), jnp.float32),
        pltpu.SemaphoreType.DMA,
        pltpu.SemaphoreType.DMA,
        pltpu.SemaphoreType.DMA,
        pltpu.SemaphoreType.DMA,
    ],
)
def _sc_decode(u_hbm, p_hbm, ti_hbm, tj_hbm, tk_hbm, ug_out, pg_out,
               iv, jv, kv, pidx, urows, prows, gsem0, gsem1, osem0, osem1):
    c = lax.axis_index("c")
    s = lax.axis_index("s")
    wid = s * 2 + c
    nchunk = T_N // (32 * 128)
    gsems = (gsem0, gsem1)
    osems = (osem0, osem1)

    def base_of(ch):
        return (wid * nchunk + ch) * 128

    def fire(ch):
        slot = ch % 2
        base = base_of(ch)
        pltpu.sync_copy(ti_hbm.at[pl.ds(base, 128)], iv.at[slot])
        pltpu.sync_copy(tj_hbm.at[pl.ds(base, 128)], jv)
        pltpu.sync_copy(tk_hbm.at[pl.ds(base, 128)], kv)
        for m in range(8):
            pidx[slot, pl.ds(m * 16, 16)] = (jv[pl.ds(m * 16, 16)] * K_T
                                             + kv[pl.ds(m * 16, 16)])
        pltpu.async_copy(u_hbm.at[iv.at[slot]], urows.at[slot], gsems[slot])
        pltpu.async_copy(p_hbm.at[pidx.at[slot]], prows.at[slot], gsems[slot])

    def wait_gather(ch):
        slot = ch % 2
        pltpu.make_async_copy(u_hbm.at[iv.at[slot]], urows.at[slot],
                              gsems[slot]).wait()
        pltpu.make_async_copy(p_hbm.at[pidx.at[slot]], prows.at[slot],
                              gsems[slot]).wait()

    def fire_out(ch):
        slot = ch % 2
        base = base_of(ch)
        pltpu.async_copy(urows.at[slot], ug_out.at[pl.ds(base, 128)],
                         osems[slot])
        pltpu.async_copy(prows.at[slot], pg_out.at[pl.ds(base, 128)],
                         osems[slot])

    def wait_out(ch):
        slot = ch % 2
        base = base_of(ch)
        pltpu.make_async_copy(urows.at[slot], ug_out.at[pl.ds(base, 128)],
                              osems[slot]).wait()
        pltpu.make_async_copy(prows.at[slot], pg_out.at[pl.ds(base, 128)],
                              osems[slot]).wait()

    fire(0)
    for ch in range(nchunk):
        if ch + 1 < nchunk:
            if ch >= 1:
                wait_out(ch - 1)
            fire(ch + 1)
        wait_gather(ch)
        fire_out(ch)
    wait_out(nchunk - 2)
    wait_out(nchunk - 1)


def _fin_body(ug_ref, pg_ref, b_ref, out_ref):
    x = jnp.sum(ug_ref[...] * pg_ref[...], axis=1) + b_ref[0]
    out_ref[...] = 1.0 / (1.0 + jnp.exp(-x))


_tc_finish = pl.pallas_call(
    _fin_body,
    grid=(8,),
    in_specs=[
        pl.BlockSpec((T_N // 8, DIM), lambda i: (i, 0)),
        pl.BlockSpec((T_N // 8, DIM), lambda i: (i, 0)),
        pl.BlockSpec(memory_space=pltpu.SMEM),
    ],
    out_specs=pl.BlockSpec((T_N // 8,), lambda i: (i,)),
    out_shape=jax.ShapeDtypeStruct((T_N,), jnp.float32),
)


def kernel(edge_index, edge_x, idx_ijk, u0_weight, w0, edge_proj_W,
           edge_proj_b, u_up_W, u_up_b, w_up_W, w_up_b, pos, in_proj_W,
           in_proj_b, out_proj_W, out_proj_b, ffn_W1, ffn_b1, ffn_W2,
           ffn_b2, ln1_g, ln1_b, ln2_g, ln2_b, V_weight, bias):
    pad = E_PAD - E_N
    ei = edge_index[0].astype(jnp.int32)
    ek = edge_index[1].astype(jnp.int32)
    ei2d = jnp.concatenate([ei, jnp.full((pad,), I_N, jnp.int32)]).reshape(
        E_PAD // 128, 128)
    ek2d = jnp.concatenate([ek, jnp.full((pad,), K_T, jnp.int32)]).reshape(
        E_PAD // 128, 128)
    z1d = jnp.zeros((ACHUNK // 8,), jnp.float32)
    z2d = jnp.zeros((EXI_R // 16, J_F), jnp.float32)

    out_a, out_exi = _sc_build(ei2d, ek2d, edge_x, z1d, z2d)
    exk = _exk_call(ek[:, None], edge_x)
    A3 = out_a.reshape(2, RHP, K_T)
    exi0 = out_exi[0:I_N]
    exi1 = out_exi[EXI_R:EXI_R + I_N]

    U, P = _tc_dense(A3, exi0, exi1, exk, u0_weight, w0, edge_proj_W,
                     edge_proj_b, u_up_W, u_up_b, w_up_W, w_up_b, pos,
                     in_proj_W, in_proj_b, out_proj_W, out_proj_b, ffn_W1,
                     ffn_b1, ffn_W2, ffn_b2, ln1_g, ln1_b, ln2_g, ln2_b,
                     V_weight)

    ti = idx_ijk[:, 0].astype(jnp.int32)
    tj = idx_ijk[:, 1].astype(jnp.int32)
    tk = idx_ijk[:, 2].astype(jnp.int32)
    ug, pg = _sc_decode(U, P, ti, tj, tk)
    return _tc_finish(ug, pg, bias.astype(jnp.float32))


# speedup vs baseline: 1.4801x; 1.0126x over previous
import functools

import jax
import jax.numpy as jnp
from jax import lax
from jax.experimental import pallas as pl
from jax.experimental.pallas import tpu as pltpu
from jax.experimental.pallas import tpu_sc as plsc

I_N, J_F, K_T, DIM, LAYERS, HEADS = 10000, 16, 256, 128, 2, 2
E_N, T_N = 320000, 65536
HD = DIM // HEADS

E_PAD = 327680
N1 = E_PAD // (16 * 128)
N2 = E_PAD // (32 * 128)
RH = I_N // 2
RHP = 5008
AFLAT = RHP * K_T
DUMMY_A = RH * K_T
ACHUNK = AFLAT // 16
EXI_R = 10240
EXK_R = 512

_mesh = plsc.VectorSubcoreMesh(core_axis_name="c", subcore_axis_name="s")
_sc_params = pltpu.CompilerParams(use_tc_tiling_on_sc=False,
                                  needs_layout_passes=False)


NR2 = E_N // 128
NCH1 = N1 // 8
NCH2 = 20


@functools.partial(
    pl.kernel,
    out_type=[
        jax.ShapeDtypeStruct((2 * AFLAT,), jnp.float32),
        jax.ShapeDtypeStruct((2 * EXI_R, J_F), jnp.float32),
    ],
    mesh=_mesh,
    compiler_params=_sc_params,
    scratch_types=[
        pltpu.VMEM_SHARED((AFLAT,), jnp.float32),
        pltpu.VMEM_SHARED((EXI_R, J_F), jnp.float32),
        pltpu.VMEM((2, 8, 128), jnp.int32),
        pltpu.VMEM((2, 8, 128), jnp.int32),
        pltpu.VMEM((2, 16, 128), jnp.int32),
        pltpu.VMEM((128,), jnp.float32),
        pltpu.VMEM((2, 4, 128), jnp.int32),
        pltpu.VMEM((2, 512, J_F), jnp.float32),
        pltpu.VMEM((ACHUNK // 16, ), jnp.float32),
        pltpu.VMEM((EXI_R // 32, J_F), jnp.float32),
        pltpu.SemaphoreType.DMA,
        pltpu.SemaphoreType.DMA,
        pltpu.SemaphoreType.DMA,
        pltpu.SemaphoreType.DMA,
        pltpu.SemaphoreType.DMA,
        pltpu.SemaphoreType.DMA,
        pltpu.SemaphoreType.DMA,
        pltpu.SemaphoreType.DMA,
    ],
)
def _sc_build(ei2d, ek2d, ei1d, ex, z1bf, z2d, ones_hbm, out_a, out_exi,
              a_sh, exi_sh, eiv, ekv, idx, ones_v, riv, exv, bnc, bexi,
              is0, is1, is2, is3, ss0, ss1, ss2, ss3):
    c = lax.axis_index("c")
    s = lax.axis_index("s")
    wid = s * 2 + c
    isems = (is0, is1, is2, is3)
    ssems = (ss0, ss1, ss2, ss3)

    pltpu.sync_copy(z1bf, bnc)
    pltpu.sync_copy(z2d, bexi)
    pltpu.sync_copy(ones_hbm, ones_v)
    for t in range(16):
        pltpu.sync_copy(bnc, a_sh.at[pl.ds(s * ACHUNK + t * (ACHUNK // 16),
                                           ACHUNK // 16)])
    for t in range(2):
        pltpu.sync_copy(bexi, exi_sh.at[pl.ds(s * (EXI_R // 16)
                                              + t * (EXI_R // 32),
                                              EXI_R // 32)])
    plsc.subcore_barrier()

    rowbase = c * RH

    def p1_fire(t, par):
        rb = s * N1 + t * 8
        pltpu.async_copy(ei2d.at[pl.ds(rb, 8)], eiv.at[par], isems[par])
        pltpu.async_copy(ek2d.at[pl.ds(rb, 8)], ekv.at[par], isems[par])

    def p1_work(t, par):
        rb = s * N1 + t * 8
        pltpu.make_async_copy(ei2d.at[pl.ds(rb, 8)], eiv.at[par],
                              isems[par]).wait()
        pltpu.make_async_copy(ek2d.at[pl.ds(rb, 8)], ekv.at[par],
                              isems[par]).wait()
        for r in range(8):
            for m in range(8):
                vi = eiv[par, r, pl.ds(m * 16, 16)]
                vk = ekv[par, r, pl.ds(m * 16, 16)]
                loc = vi - rowbase
                ok = (loc >= 0) & (loc < RH)
                idx[par, r, pl.ds(m * 16, 16)] = jnp.where(
                    ok, loc * K_T + vk, DUMMY_A)
        for r in range(8):
            pltpu.async_copy(ones_v, a_sh.at[idx.at[par].at[r]],
                             ssems[par], add=True)

    def p1_drain(par):
        for r in range(8):
            pltpu.make_async_copy(ones_v, a_sh.at[idx.at[par].at[r]],
                                  ssems[par]).wait()

    p1_fire(0, 0)
    p1_fire(1, 1)

    def p1_body(t2, carry):
        for par in range(2):
            t = t2 * 2 + par

            @pl.when(t2 > 0)
            def _():
                p1_drain(par)

            p1_work(t, par)

            @pl.when(t + 2 < NCH1)
            def _():
                p1_fire(t + 2, par)
        return carry

    lax.fori_loop(0, NCH1 // 2, p1_body, 0)
    p1_drain(0)
    p1_drain(1)

    def p2_fire_in(t, par):
        rb = wid * N2 + t * 4

        @pl.when(rb < NR2)
        def _():
            pltpu.async_copy(ei2d.at[pl.ds(rb, 4)], riv.at[par],
                             isems[par])
            pltpu.async_copy(ex.at[pl.ds(rb * 128, 512)], exv.at[par],
                             isems[par])

    def p2_wait_in(t, par):
        rb = wid * N2 + t * 4

        @pl.when(rb < NR2)
        def _():
            pltpu.make_async_copy(ei2d.at[pl.ds(rb, 4)], riv.at[par],
                                  isems[par]).wait()
            pltpu.make_async_copy(ex.at[pl.ds(rb * 128, 512)], exv.at[par],
                                  isems[par]).wait()

    def p2_fire_sc(t, par):
        rb = wid * N2 + t * 4

        @pl.when(rb < NR2)
        def _():
            for r in range(4):
                pltpu.async_copy(exv.at[par].at[pl.ds(r * 128, 128)],
                                 exi_sh.at[riv.at[par].at[r]],
                                 ssems[par], add=True)

    def p2_drain_sc(t, par):
        rb = wid * N2 + t * 4

        @pl.when(rb < NR2)
        def _():
            for r in range(4):
                pltpu.make_async_copy(exv.at[par].at[pl.ds(r * 128, 128)],
                                     exi_sh.at[riv.at[par].at[r]],
                                     ssems[par]).wait()

    p2_fire_in(0, 0)

    def p2_body(t2, carry):
        for par in range(2):
            t = t2 * 2 + par
            p2_wait_in(t, par)
            if par == 0:
                @pl.when(t2 > 0)
                def _():
                    p2_drain_sc(t - 1, 1)
            else:
                p2_drain_sc(t - 1, 0)
            p2_fire_sc(t, par)
            if par == 0:
                p2_fire_in(t + 1, 1)
            else:
                @pl.when(t2 < NCH2 // 2 - 1)
                def _():
                    p2_fire_in(t + 1, 0)
        return carry

    lax.fori_loop(0, NCH2 // 2, p2_body, 0)
    p2_drain_sc(NCH2 - 1, 1)
    plsc.subcore_barrier()

    for t in range(16):
        off = s * ACHUNK + t * (ACHUNK // 16)
        pltpu.sync_copy(a_sh.at[pl.ds(off, ACHUNK // 16)], bnc)
        pltpu.sync_copy(bnc, out_a.at[pl.ds(c * AFLAT + off, ACHUNK // 16)])
    for t in range(2):
        off = s * (EXI_R // 16) + t * (EXI_R // 32)
        pltpu.sync_copy(exi_sh.at[pl.ds(off, EXI_R // 32)], bexi)
        pltpu.sync_copy(bexi, out_exi.at[pl.ds(c * EXI_R + off, EXI_R // 32)])


_EXK_CH = 2560


def _exk_body(ek_ref, ex_ref, o_ref):
    i = pl.program_id(0)
    oh = (lax.broadcasted_iota(jnp.int32, (_EXK_CH, K_T), 1)
          == ek_ref[...]).astype(jnp.float32)
    acc = lax.dot_general(oh, ex_ref[...], (((0,), (0,)), ((), ())),
                          preferred_element_type=jnp.float32)

    @pl.when(i == 0)
    def _():
        o_ref[...] = acc

    @pl.when(i > 0)
    def _():
        o_ref[...] += acc


_exk_call = pl.pallas_call(
    _exk_body,
    grid=(E_N // _EXK_CH,),
    in_specs=[
        pl.BlockSpec((_EXK_CH, 1), lambda i: (i, 0)),
        pl.BlockSpec((_EXK_CH, J_F), lambda i: (i, 0)),
    ],
    out_specs=pl.BlockSpec((K_T, J_F), lambda i: (0, 0)),
    out_shape=jax.ShapeDtypeStruct((K_T, J_F), jnp.float32),
)


def _mmt(x, w):
    return lax.dot_general(x, w, (((1,), (1,)), ((), ())),
                           preferred_element_type=jnp.float32)


def _layer_norm(x, g, b):
    m = jnp.mean(x, axis=-1, keepdims=True)
    v = jnp.mean((x - m) ** 2, axis=-1, keepdims=True)
    return (x - m) * lax.rsqrt(v + 1e-5) * g + b


def _tc_body(a_ref, exi0_ref, exi1_ref, exk_ref, u0_ref, w0_ref,
             epw_ref, epb_ref, uupw_ref, uupb_ref, wupw_ref, wupb_ref,
             pos_ref, inw_ref, inb_ref, outw_ref, outb_ref, f1w_ref, f1b_ref,
             f2w_ref, f2b_ref, l1g_ref, l1b_ref, l2g_ref, l2b_ref, v_ref,
             u_out_ref, p_out_ref):
    A = jnp.concatenate([a_ref[0, :RH, :], a_ref[1, :RH, :]],
                        axis=0).astype(jnp.float32)
    exi = exi0_ref[...] + exi1_ref[...]
    exk = exk_ref[...]
    cnt_u_raw = jnp.sum(A, axis=1, keepdims=True)
    cnt_w_raw = jnp.sum(A, axis=0)[:, None]
    cnt_u = jnp.maximum(cnt_u_raw, 1.0)
    cnt_w = jnp.maximum(cnt_w_raw, 1.0)
    epb = epb_ref[...]
    eproj_i = _mmt(exi, epw_ref[...]) + cnt_u_raw * epb[None, :]
    eproj_k = _mmt(exk, epw_ref[...]) + cnt_w_raw * epb[None, :]

    U = u0_ref[...]
    W = w0_ref[...]
    for l in range(LAYERS):
        u_msg = lax.dot_general(A, W, (((1,), (0,)), ((), ())),
                                preferred_element_type=jnp.float32) + eproj_i
        w_msg = lax.dot_general(A, U, (((0,), (0,)), ((), ())),
                                preferred_element_type=jnp.float32) + eproj_k
        uw = uupw_ref[l]
        ww = wupw_ref[l]
        U = jax.nn.relu(_mmt(U, uw[:, :DIM]) + _mmt(u_msg / cnt_u, uw[:, DIM:])
                        + uupb_ref[l][None, :])
        W = jax.nn.relu(_mmt(W, ww[:, :DIM]) + _mmt(w_msg / cnt_w, ww[:, DIM:])
                        + wupb_ref[l][None, :])

    X = W + pos_ref[...]
    qkv = _mmt(X, inw_ref[...]) + inb_ref[...][None, :]
    aos = []
    for h in range(HEADS):
        qh = qkv[:, h * HD:(h + 1) * HD]
        kh = qkv[:, DIM + h * HD:DIM + (h + 1) * HD]
        vh = qkv[:, 2 * DIM + h * HD:2 * DIM + (h + 1) * HD]
        logits = _mmt(qh, kh) * (1.0 / (HD ** 0.5))
        logits = logits - jnp.max(logits, axis=-1, keepdims=True)
        e = jnp.exp(logits)
        attn = e / jnp.sum(e, axis=-1, keepdims=True)
        aos.append(lax.dot_general(attn, vh, (((1,), (0,)), ((), ())),
                                   preferred_element_type=jnp.float32))
    ao = _mmt(jnp.concatenate(aos, axis=1), outw_ref[...]) + outb_ref[...][None, :]
    H1 = _layer_norm(X + ao, l1g_ref[...], l1b_ref[...])
    H2 = _mmt(jax.nn.relu(_mmt(H1, f1w_ref[...]) + f1b_ref[...][None, :]),
              f2w_ref[...]) + f2b_ref[...][None, :]
    Wr = _layer_norm(H1 + H2, l2g_ref[...], l2b_ref[...])

    u_out_ref[...] = U
    p_out_ref[...] = (v_ref[...][:, None, :] * Wr[None, :, :]).reshape(
        J_F * K_T, DIM)


_tc_dense = pl.pallas_call(
    _tc_body,
    out_shape=[
        jax.ShapeDtypeStruct((I_N, DIM), jnp.float32),
        jax.ShapeDtypeStruct((J_F * K_T, DIM), jnp.float32),
    ],
)


@functools.partial(
    pl.kernel,
    out_type=[
        jax.ShapeDtypeStruct((T_N, DIM), jnp.float32),
        jax.ShapeDtypeStruct((T_N, DIM), jnp.float32),
    ],
    mesh=_mesh,
    compiler_params=_sc_params,
    scratch_types=[
        pltpu.VMEM((2, 128), jnp.int32),
        pltpu.VMEM((128,), jnp.int32),
        pltpu.VMEM((128,), jnp.int32),
        pltpu.VMEM((2, 128), jnp.int32),
        pltpu.VMEM((2, 128, DIM), jnp.float32),
        pltpu.VMEM((2, 128, DIM), jnp.float32),
        pltpu.SemaphoreType.DMA,
        pltpu.SemaphoreType.DMA,
        pltpu.SemaphoreType.DMA,
        pltpu.SemaphoreType.DMA,
    ],
)
def _sc_decode(u_hbm, p_hbm, ti_hbm, tj_hbm, tk_hbm, ug_out, pg_out,
               iv, jv, kv, pidx, urows, prows, gsem0, gsem1, osem0, osem1):
    c = lax.axis_index("c")
    s = lax.axis_index("s")
    wid = s * 2 + c
    nchunk = T_N // (32 * 128)
    gsems = (gsem0, gsem1)
    osems = (osem0, osem1)

    def base_of(ch):
        return (wid * nchunk + ch) * 128

    def fire(ch):
        slot = ch % 2
        base = base_of(ch)
        pltpu.sync_copy(ti_hbm.at[pl.ds(base, 128)], iv.at[slot])
        pltpu.sync_copy(tj_hbm.at[pl.ds(base, 128)], jv)
        pltpu.sync_copy(tk_hbm.at[pl.ds(base, 128)], kv)
        for m in range(8):
            pidx[slot, pl.ds(m * 16, 16)] = (jv[pl.ds(m * 16, 16)] * K_T
                                             + kv[pl.ds(m * 16, 16)])
        pltpu.async_copy(u_hbm.at[iv.at[slot]], urows.at[slot], gsems[slot])
        pltpu.async_copy(p_hbm.at[pidx.at[slot]], prows.at[slot], gsems[slot])

    def wait_gather(ch):
        slot = ch % 2
        pltpu.make_async_copy(u_hbm.at[iv.at[slot]], urows.at[slot],
                              gsems[slot]).wait()
        pltpu.make_async_copy(p_hbm.at[pidx.at[slot]], prows.at[slot],
                              gsems[slot]).wait()

    def fire_out(ch):
        slot = ch % 2
        base = base_of(ch)
        pltpu.async_copy(urows.at[slot], ug_out.at[pl.ds(base, 128)],
                         osems[slot])
        pltpu.async_copy(prows.at[slot], pg_out.at[pl.ds(base, 128)],
                         osems[slot])

    def wait_out(ch):
        slot = ch % 2
        base = base_of(ch)
        pltpu.make_async_copy(urows.at[slot], ug_out.at[pl.ds(base, 128)],
                              osems[slot]).wait()
        pltpu.make_async_copy(prows.at[slot], pg_out.at[pl.ds(base, 128)],
                              osems[slot]).wait()

    fire(0)
    for ch in range(nchunk):
        if ch + 1 < nchunk:
            if ch >= 1:
                wait_out(ch - 1)
            fire(ch + 1)
        wait_gather(ch)
        fire_out(ch)
    wait_out(nchunk - 2)
    wait_out(nchunk - 1)


def _fin_body(ug_ref, pg_ref, b_ref, out_ref):
    x = jnp.sum(ug_ref[...] * pg_ref[...], axis=1) + b_ref[0]
    out_ref[...] = 1.0 / (1.0 + jnp.exp(-x))


_tc_finish = pl.pallas_call(
    _fin_body,
    grid=(8,),
    in_specs=[
        pl.BlockSpec((T_N // 8, DIM), lambda i: (i, 0)),
        pl.BlockSpec((T_N // 8, DIM), lambda i: (i, 0)),
        pl.BlockSpec(memory_space=pltpu.SMEM),
    ],
    out_specs=pl.BlockSpec((T_N // 8,), lambda i: (i,)),
    out_shape=jax.ShapeDtypeStruct((T_N,), jnp.float32),
)


def kernel(edge_index, edge_x, idx_ijk, u0_weight, w0, edge_proj_W,
           edge_proj_b, u_up_W, u_up_b, w_up_W, w_up_b, pos, in_proj_W,
           in_proj_b, out_proj_W, out_proj_b, ffn_W1, ffn_b1, ffn_W2,
           ffn_b2, ln1_g, ln1_b, ln2_g, ln2_b, V_weight, bias):
    pad = E_PAD - E_N
    ei = edge_index[0].astype(jnp.int32)
    ek = edge_index[1].astype(jnp.int32)
    ei2d = jnp.concatenate([ei, jnp.full((pad,), I_N, jnp.int32)]).reshape(
        E_PAD // 128, 128)
    ek2d = jnp.concatenate([ek, jnp.full((pad,), K_T, jnp.int32)]).reshape(
        E_PAD // 128, 128)
    z1bf = jnp.zeros((ACHUNK // 16,), jnp.float32)
    z2d = jnp.zeros((EXI_R // 32, J_F), jnp.float32)
    ones2048 = jnp.ones((128,), jnp.float32)

    ei1d = jnp.concatenate([ei, jnp.zeros((pad,), jnp.int32)])
    out_a, out_exi = _sc_build(ei2d, ek2d, ei1d, edge_x, z1bf, z2d, ones2048)
    exk = _exk_call(ek[:, None], edge_x)
    A3 = out_a.reshape(2, RHP, K_T)
    exi0 = out_exi[0:I_N]
    exi1 = out_exi[EXI_R:EXI_R + I_N]

    U, P = _tc_dense(A3, exi0, exi1, exk, u0_weight, w0, edge_proj_W,
                     edge_proj_b, u_up_W, u_up_b, w_up_W, w_up_b, pos,
                     in_proj_W, in_proj_b, out_proj_W, out_proj_b, ffn_W1,
                     ffn_b1, ffn_W2, ffn_b2, ln1_g, ln1_b, ln2_g, ln2_b,
                     V_weight)

    ti = idx_ijk[:, 0].astype(jnp.int32)
    tj = idx_ijk[:, 1].astype(jnp.int32)
    tk = idx_ijk[:, 2].astype(jnp.int32)
    ug, pg = _sc_decode(U, P, ti, tj, tk)
    return _tc_finish(ug, pg, bias.astype(jnp.float32))


# speedup vs baseline: 1.4837x; 1.0024x over previous
import functools

import jax
import jax.numpy as jnp
from jax import lax
from jax.experimental import pallas as pl
from jax.experimental.pallas import tpu as pltpu
from jax.experimental.pallas import tpu_sc as plsc

I_N, J_F, K_T, DIM, LAYERS, HEADS = 10000, 16, 256, 128, 2, 2
E_N, T_N = 320000, 65536
HD = DIM // HEADS

E_PAD = 327680
N1 = E_PAD // (16 * 128)
N2 = E_PAD // (32 * 128)
RH = I_N // 2
RHP = 5008
AFLAT = RHP * K_T
DUMMY_A = RH * K_T
ACHUNK = AFLAT // 16
EXI_R = 10240
EXK_R = 512

_mesh = plsc.VectorSubcoreMesh(core_axis_name="c", subcore_axis_name="s")
_sc_params = pltpu.CompilerParams(use_tc_tiling_on_sc=False,
                                  needs_layout_passes=False)


NR2 = E_N // 128
NCH1 = N1 // 8
NCH2 = 20


@functools.partial(
    pl.kernel,
    out_type=jax.ShapeDtypeStruct((2 * AFLAT,), jnp.float32),
    mesh=_mesh,
    compiler_params=_sc_params,
    scratch_types=[
        pltpu.VMEM_SHARED((AFLAT,), jnp.float32),
        pltpu.VMEM((2, 8, 128), jnp.int32),
        pltpu.VMEM((2, 8, 128), jnp.int32),
        pltpu.VMEM((2, 16, 128), jnp.int32),
        pltpu.VMEM((128,), jnp.float32),
        pltpu.VMEM((ACHUNK // 16, ), jnp.float32),
        pltpu.SemaphoreType.DMA,
        pltpu.SemaphoreType.DMA,
        pltpu.SemaphoreType.DMA,
        pltpu.SemaphoreType.DMA,
    ],
)
def _sc_build_a(ei2d, ek2d, z1bf, ones_hbm, out_a,
                a_sh, eiv, ekv, idx, ones_v, bnc,
                is0, is1, ss0, ss1):
    c = lax.axis_index("c")
    s = lax.axis_index("s")
    isems = (is0, is1)
    ssems = (ss0, ss1)

    pltpu.sync_copy(z1bf, bnc)
    pltpu.sync_copy(ones_hbm, ones_v)
    for t in range(16):
        pltpu.sync_copy(bnc, a_sh.at[pl.ds(s * ACHUNK + t * (ACHUNK // 16),
                                           ACHUNK // 16)])
    plsc.subcore_barrier()

    rowbase = c * RH

    def p1_fire(t, par):
        rb = s * N1 + t * 8
        pltpu.async_copy(ei2d.at[pl.ds(rb, 8)], eiv.at[par], isems[par])
        pltpu.async_copy(ek2d.at[pl.ds(rb, 8)], ekv.at[par], isems[par])

    def p1_work(t, par):
        rb = s * N1 + t * 8
        pltpu.make_async_copy(ei2d.at[pl.ds(rb, 8)], eiv.at[par],
                              isems[par]).wait()
        pltpu.make_async_copy(ek2d.at[pl.ds(rb, 8)], ekv.at[par],
                              isems[par]).wait()
        for r in range(8):
            for m in range(8):
                vi = eiv[par, r, pl.ds(m * 16, 16)]
                vk = ekv[par, r, pl.ds(m * 16, 16)]
                loc = vi - rowbase
                ok = (loc >= 0) & (loc < RH)
                idx[par, r, pl.ds(m * 16, 16)] = jnp.where(
                    ok, loc * K_T + vk, DUMMY_A)
        for r in range(8):
            pltpu.async_copy(ones_v, a_sh.at[idx.at[par].at[r]],
                             ssems[par], add=True)

    def p1_drain(par):
        for r in range(8):
            pltpu.make_async_copy(ones_v, a_sh.at[idx.at[par].at[r]],
                                  ssems[par]).wait()

    p1_fire(0, 0)
    p1_fire(1, 1)

    def p1_body(t2, carry):
        for par in range(2):
            t = t2 * 2 + par

            @pl.when(t2 > 0)
            def _():
                p1_drain(par)

            p1_work(t, par)

            @pl.when(t + 2 < NCH1)
            def _():
                p1_fire(t + 2, par)
        return carry

    lax.fori_loop(0, NCH1 // 2, p1_body, 0)
    p1_drain(0)
    p1_drain(1)
    plsc.subcore_barrier()

    for t in range(16):
        off = s * ACHUNK + t * (ACHUNK // 16)
        pltpu.sync_copy(a_sh.at[pl.ds(off, ACHUNK // 16)], bnc)
        pltpu.sync_copy(bnc, out_a.at[pl.ds(c * AFLAT + off, ACHUNK // 16)])


@functools.partial(
    pl.kernel,
    out_type=jax.ShapeDtypeStruct((2 * EXI_R, J_F), jnp.float32),
    mesh=_mesh,
    compiler_params=_sc_params,
    scratch_types=[
        pltpu.VMEM_SHARED((EXI_R, J_F), jnp.float32),
        pltpu.VMEM((2, 4, 128), jnp.int32),
        pltpu.VMEM((2, 512, J_F), jnp.float32),
        pltpu.VMEM((EXI_R // 32, J_F), jnp.float32),
        pltpu.SemaphoreType.DMA,
        pltpu.SemaphoreType.DMA,
        pltpu.SemaphoreType.DMA,
        pltpu.SemaphoreType.DMA,
    ],
)
def _sc_build_exi(ei2d, ex, z2d, out_exi,
                  exi_sh, riv, exv, bexi,
                  is0, is1, ss0, ss1):
    c = lax.axis_index("c")
    s = lax.axis_index("s")
    wid = s * 2 + c
    isems = (is0, is1)
    ssems = (ss0, ss1)

    pltpu.sync_copy(z2d, bexi)
    for t in range(2):
        pltpu.sync_copy(bexi, exi_sh.at[pl.ds(s * (EXI_R // 16)
                                              + t * (EXI_R // 32),
                                              EXI_R // 32)])
    plsc.subcore_barrier()

    def p2_fire_in(t, par):
        rb = wid * N2 + t * 4

        @pl.when(rb < NR2)
        def _():
            pltpu.async_copy(ei2d.at[pl.ds(rb, 4)], riv.at[par],
                             isems[par])
            pltpu.async_copy(ex.at[pl.ds(rb * 128, 512)], exv.at[par],
                             isems[par])

    def p2_wait_in(t, par):
        rb = wid * N2 + t * 4

        @pl.when(rb < NR2)
        def _():
            pltpu.make_async_copy(ei2d.at[pl.ds(rb, 4)], riv.at[par],
                                  isems[par]).wait()
            pltpu.make_async_copy(ex.at[pl.ds(rb * 128, 512)], exv.at[par],
                                  isems[par]).wait()

    def p2_fire_sc(t, par):
        rb = wid * N2 + t * 4

        @pl.when(rb < NR2)
        def _():
            for r in range(4):
                pltpu.async_copy(exv.at[par].at[pl.ds(r * 128, 128)],
                                 exi_sh.at[riv.at[par].at[r]],
                                 ssems[par], add=True)

    def p2_drain_sc(t, par):
        rb = wid * N2 + t * 4

        @pl.when(rb < NR2)
        def _():
            for r in range(4):
                pltpu.make_async_copy(exv.at[par].at[pl.ds(r * 128, 128)],
                                     exi_sh.at[riv.at[par].at[r]],
                                     ssems[par]).wait()

    p2_fire_in(0, 0)

    def p2_body(t2, carry):
        for par in range(2):
            t = t2 * 2 + par
            p2_wait_in(t, par)
            if par == 0:
                @pl.when(t2 > 0)
                def _():
                    p2_drain_sc(t - 1, 1)
            else:
                p2_drain_sc(t - 1, 0)
            p2_fire_sc(t, par)
            if par == 0:
                p2_fire_in(t + 1, 1)
            else:
                @pl.when(t2 < NCH2 // 2 - 1)
                def _():
                    p2_fire_in(t + 1, 0)
        return carry

    lax.fori_loop(0, NCH2 // 2, p2_body, 0)
    p2_drain_sc(NCH2 - 1, 1)
    plsc.subcore_barrier()

    for t in range(2):
        off = s * (EXI_R // 16) + t * (EXI_R // 32)
        pltpu.sync_copy(exi_sh.at[pl.ds(off, EXI_R // 32)], bexi)
        pltpu.sync_copy(bexi, out_exi.at[pl.ds(c * EXI_R + off, EXI_R // 32)])


_EXK_CH = 2560


def _exk_body(ek_ref, ex_ref, o_ref):
    i = pl.program_id(0)
    oh = (lax.broadcasted_iota(jnp.int32, (_EXK_CH, K_T), 1)
          == ek_ref[...]).astype(jnp.float32)
    acc = lax.dot_general(oh, ex_ref[...], (((0,), (0,)), ((), ())),
                          preferred_element_type=jnp.float32)

    @pl.when(i == 0)
    def _():
        o_ref[...] = acc

    @pl.when(i > 0)
    def _():
        o_ref[...] += acc


_exk_call = pl.pallas_call(
    _exk_body,
    grid=(E_N // _EXK_CH,),
    in_specs=[
        pl.BlockSpec((_EXK_CH, 1), lambda i: (i, 0)),
        pl.BlockSpec((_EXK_CH, J_F), lambda i: (i, 0)),
    ],
    out_specs=pl.BlockSpec((K_T, J_F), lambda i: (0, 0)),
    out_shape=jax.ShapeDtypeStruct((K_T, J_F), jnp.float32),
)


def _mmt(x, w):
    return lax.dot_general(x, w, (((1,), (1,)), ((), ())),
                           preferred_element_type=jnp.float32)


def _layer_norm(x, g, b):
    m = jnp.mean(x, axis=-1, keepdims=True)
    v = jnp.mean((x - m) ** 2, axis=-1, keepdims=True)
    return (x - m) * lax.rsqrt(v + 1e-5) * g + b


def _tc_body(a_ref, exi0_ref, exi1_ref, exk_ref, u0_ref, w0_ref,
             epw_ref, epb_ref, uupw_ref, uupb_ref, wupw_ref, wupb_ref,
             pos_ref, inw_ref, inb_ref, outw_ref, outb_ref, f1w_ref, f1b_ref,
             f2w_ref, f2b_ref, l1g_ref, l1b_ref, l2g_ref, l2b_ref, v_ref,
             u_out_ref, p_out_ref):
    A = jnp.concatenate([a_ref[0, :RH, :], a_ref[1, :RH, :]],
                        axis=0).astype(jnp.float32)
    exi = exi0_ref[...] + exi1_ref[...]
    exk = exk_ref[...]
    cnt_u_raw = jnp.sum(A, axis=1, keepdims=True)
    cnt_w_raw = jnp.sum(A, axis=0)[:, None]
    cnt_u = jnp.maximum(cnt_u_raw, 1.0)
    cnt_w = jnp.maximum(cnt_w_raw, 1.0)
    epb = epb_ref[...]
    eproj_i = _mmt(exi, epw_ref[...]) + cnt_u_raw * epb[None, :]
    eproj_k = _mmt(exk, epw_ref[...]) + cnt_w_raw * epb[None, :]

    U = u0_ref[...]
    W = w0_ref[...]
    for l in range(LAYERS):
        u_msg = lax.dot_general(A, W, (((1,), (0,)), ((), ())),
                                preferred_element_type=jnp.float32) + eproj_i
        w_msg = lax.dot_general(A, U, (((0,), (0,)), ((), ())),
                                preferred_element_type=jnp.float32) + eproj_k
        uw = uupw_ref[l]
        ww = wupw_ref[l]
        U = jax.nn.relu(_mmt(U, uw[:, :DIM]) + _mmt(u_msg / cnt_u, uw[:, DIM:])
                        + uupb_ref[l][None, :])
        W = jax.nn.relu(_mmt(W, ww[:, :DIM]) + _mmt(w_msg / cnt_w, ww[:, DIM:])
                        + wupb_ref[l][None, :])

    X = W + pos_ref[...]
    qkv = _mmt(X, inw_ref[...]) + inb_ref[...][None, :]
    aos = []
    for h in range(HEADS):
        qh = qkv[:, h * HD:(h + 1) * HD]
        kh = qkv[:, DIM + h * HD:DIM + (h + 1) * HD]
        vh = qkv[:, 2 * DIM + h * HD:2 * DIM + (h + 1) * HD]
        logits = _mmt(qh, kh) * (1.0 / (HD ** 0.5))
        logits = logits - jnp.max(logits, axis=-1, keepdims=True)
        e = jnp.exp(logits)
        attn = e / jnp.sum(e, axis=-1, keepdims=True)
        aos.append(lax.dot_general(attn, vh, (((1,), (0,)), ((), ())),
                                   preferred_element_type=jnp.float32))
    ao = _mmt(jnp.concatenate(aos, axis=1), outw_ref[...]) + outb_ref[...][None, :]
    H1 = _layer_norm(X + ao, l1g_ref[...], l1b_ref[...])
    H2 = _mmt(jax.nn.relu(_mmt(H1, f1w_ref[...]) + f1b_ref[...][None, :]),
              f2w_ref[...]) + f2b_ref[...][None, :]
    Wr = _layer_norm(H1 + H2, l2g_ref[...], l2b_ref[...])

    u_out_ref[...] = U
    p_out_ref[...] = (v_ref[...][:, None, :] * Wr[None, :, :]).reshape(
        J_F * K_T, DIM)


_tc_dense = pl.pallas_call(
    _tc_body,
    out_shape=[
        jax.ShapeDtypeStruct((I_N, DIM), jnp.float32),
        jax.ShapeDtypeStruct((J_F * K_T, DIM), jnp.float32),
    ],
)


@functools.partial(
    pl.kernel,
    out_type=[
        jax.ShapeDtypeStruct((T_N, DIM), jnp.float32),
        jax.ShapeDtypeStruct((T_N, DIM), jnp.float32),
    ],
    mesh=_mesh,
    compiler_params=_sc_params,
    scratch_types=[
        pltpu.VMEM((2, 128), jnp.int32),
        pltpu.VMEM((128,), jnp.int32),
        pltpu.VMEM((128,), jnp.int32),
        pltpu.VMEM((2, 128), jnp.int32),
        pltpu.VMEM((2, 128, DIM), jnp.float32),
        pltpu.VMEM((2, 128, DIM), jnp.float32),
        pltpu.SemaphoreType.DMA,
        pltpu.SemaphoreType.DMA,
        pltpu.SemaphoreType.DMA,
        pltpu.SemaphoreType.DMA,
    ],
)
def _sc_decode(u_hbm, p_hbm, ti_hbm, tj_hbm, tk_hbm, ug_out, pg_out,
               iv, jv, kv, pidx, urows, prows, gsem0, gsem1, osem0, osem1):
    c = lax.axis_index("c")
    s = lax.axis_index("s")
    wid = s * 2 + c
    nchunk = T_N // (32 * 128)
    gsems = (gsem0, gsem1)
    osems = (osem0, osem1)

    def base_of(ch):
        return (wid * nchunk + ch) * 128

    def fire(ch):
        slot = ch % 2
        base = base_of(ch)
        pltpu.sync_copy(ti_hbm.at[pl.ds(base, 128)], iv.at[slot])
        pltpu.sync_copy(tj_hbm.at[pl.ds(base, 128)], jv)
        pltpu.sync_copy(tk_hbm.at[pl.ds(base, 128)], kv)
        for m in range(8):
            pidx[slot, pl.ds(m * 16, 16)] = (jv[pl.ds(m * 16, 16)] * K_T
                                             + kv[pl.ds(m * 16, 16)])
        pltpu.async_copy(u_hbm.at[iv.at[slot]], urows.at[slot], gsems[slot])
        pltpu.async_copy(p_hbm.at[pidx.at[slot]], prows.at[slot], gsems[slot])

    def wait_gather(ch):
        slot = ch % 2
        pltpu.make_async_copy(u_hbm.at[iv.at[slot]], urows.at[slot],
                              gsems[slot]).wait()
        pltpu.make_async_copy(p_hbm.at[pidx.at[slot]], prows.at[slot],
                              gsems[slot]).wait()

    def fire_out(ch):
        slot = ch % 2
        base = base_of(ch)
        pltpu.async_copy(urows.at[slot], ug_out.at[pl.ds(base, 128)],
                         osems[slot])
        pltpu.async_copy(prows.at[slot], pg_out.at[pl.ds(base, 128)],
                         osems[slot])

    def wait_out(ch):
        slot = ch % 2
        base = base_of(ch)
        pltpu.make_async_copy(urows.at[slot], ug_out.at[pl.ds(base, 128)],
                              osems[slot]).wait()
        pltpu.make_async_copy(prows.at[slot], pg_out.at[pl.ds(base, 128)],
                              osems[slot]).wait()

    fire(0)
    for ch in range(nchunk):
        if ch + 1 < nchunk:
            if ch >= 1:
                wait_out(ch - 1)
            fire(ch + 1)
        wait_gather(ch)
        fire_out(ch)
    wait_out(nchunk - 2)
    wait_out(nchunk - 1)


def _fin_body(ug_ref, pg_ref, b_ref, out_ref):
    x = jnp.sum(ug_ref[...] * pg_ref[...], axis=1) + b_ref[0]
    out_ref[...] = 1.0 / (1.0 + jnp.exp(-x))


_tc_finish = pl.pallas_call(
    _fin_body,
    grid=(8,),
    in_specs=[
        pl.BlockSpec((T_N // 8, DIM), lambda i: (i, 0)),
        pl.BlockSpec((T_N // 8, DIM), lambda i: (i, 0)),
        pl.BlockSpec(memory_space=pltpu.SMEM),
    ],
    out_specs=pl.BlockSpec((T_N // 8,), lambda i: (i,)),
    out_shape=jax.ShapeDtypeStruct((T_N,), jnp.float32),
)


def kernel(edge_index, edge_x, idx_ijk, u0_weight, w0, edge_proj_W,
           edge_proj_b, u_up_W, u_up_b, w_up_W, w_up_b, pos, in_proj_W,
           in_proj_b, out_proj_W, out_proj_b, ffn_W1, ffn_b1, ffn_W2,
           ffn_b2, ln1_g, ln1_b, ln2_g, ln2_b, V_weight, bias):
    pad = E_PAD - E_N
    ei = edge_index[0].astype(jnp.int32)
    ek = edge_index[1].astype(jnp.int32)
    ei2d = jnp.concatenate([ei, jnp.full((pad,), I_N, jnp.int32)]).reshape(
        E_PAD // 128, 128)
    ek2d = jnp.concatenate([ek, jnp.full((pad,), K_T, jnp.int32)]).reshape(
        E_PAD // 128, 128)
    z1bf = jnp.zeros((ACHUNK // 16,), jnp.float32)
    z2d = jnp.zeros((EXI_R // 32, J_F), jnp.float32)
    ones2048 = jnp.ones((128,), jnp.float32)

    out_a = _sc_build_a(ei2d, ek2d, z1bf, ones2048)
    out_exi = _sc_build_exi(ei2d, edge_x, z2d)
    exk = _exk_call(ek[:, None], edge_x)
    A3 = out_a.reshape(2, RHP, K_T)
    exi0 = out_exi[0:I_N]
    exi1 = out_exi[EXI_R:EXI_R + I_N]

    U, P = _tc_dense(A3, exi0, exi1, exk, u0_weight, w0, edge_proj_W,
                     edge_proj_b, u_up_W, u_up_b, w_up_W, w_up_b, pos,
                     in_proj_W, in_proj_b, out_proj_W, out_proj_b, ffn_W1,
                     ffn_b1, ffn_W2, ffn_b2, ln1_g, ln1_b, ln2_g, ln2_b,
                     V_weight)

    ti = idx_ijk[:, 0].astype(jnp.int32)
    tj = idx_ijk[:, 1].astype(jnp.int32)
    tk = idx_ijk[:, 2].astype(jnp.int32)
    ug, pg = _sc_decode(U, P, ti, tj, tk)
    return _tc_finish(ug, pg, bias.astype(jnp.float32))
